# baseline (speedup 1.0000x reference)
import functools

import jax
import jax.numpy as jnp
from jax import lax
from jax.experimental import pallas as pl
from jax.experimental.pallas import tpu as pltpu
from jax.experimental.pallas import tpu_sc as plsc

Q_MIN = -128.0
Q_MAX = 127.0
_MAGIC = 1.5 * (2.0 ** 23)
_PRE = 1024.0
_CH = 256


@functools.cache
def _build(V, D, B):
  info = plsc.get_sparse_core_info()
  NC, NS, L = info.num_cores, info.num_subcores, info.num_lanes
  NW = NC * NS
  assert D % L == 0 and B % (8 * NW) == 0
  b_per_w = B // NW
  n_ch = b_per_w // _CH
  mesh = plsc.VectorSubcoreMesh(core_axis_name="c", subcore_axis_name="s")

  @functools.partial(
      pl.kernel,
      out_type=jax.ShapeDtypeStruct((B // 2, 2 * D), jnp.float32),
      mesh=mesh,
      scratch_types=[
          pltpu.VMEM((b_per_w + L,), jnp.int32),
          pltpu.VMEM((b_per_w,), jnp.int32),
          pltpu.VMEM((b_per_w + L,), jnp.float32),
          pltpu.VMEM((_CH, 2 * D), jnp.float32),
          pltpu.VMEM((_CH // 2, 2 * D), jnp.float32),
          pltpu.SemaphoreType.DMA,
          pltpu.SemaphoreType.DMA,
      ],
  )
  def dequant_gather(table_hbm, idx_hbm, scales_hbm, out_hbm,
                     idx_v, pair_v, sc_v, rows_v, outc_v, sem_rows, sem_sc):
    wid = lax.axis_index("s") * NC + lax.axis_index("c")
    base = wid * b_per_w
    pltpu.sync_copy(idx_hbm.at[pl.ds(base, b_per_w)],
                    idx_v.at[pl.ds(0, b_per_w)])
    sc_cp = pltpu.async_copy(scales_hbm.at[idx_v.at[pl.ds(0, b_per_w)]],
                             sc_v.at[pl.ds(0, b_per_w)], sem_sc)
    for i in range(b_per_w // L):
      pair_v[pl.ds(i * L, L)] = lax.shift_right_logical(
          idx_v[pl.ds(i * L, L)], 1)
    sc_cp.wait()

    for ch in range(n_ch):
      cbase = ch * _CH
      pltpu.async_copy(table_hbm.at[pair_v.at[pl.ds(cbase, _CH)]],
                       rows_v, sem_rows).wait()

      def row_body(i, carry, cbase=cbase):
        tok = idx_v[pl.ds(cbase + i, L)][0]
        col0 = lax.rem(tok, 2) * D
        s = sc_v[pl.ds(cbase + i, L)][0]
        half = lax.rem(i, 2) * D
        for j in range(D // L):
          v = rows_v[i, pl.ds(col0 + j * L, L)]
          v = jnp.minimum(jnp.maximum(v, -_PRE), _PRE)
          v = (v + _MAGIC) - _MAGIC
          v = jnp.minimum(jnp.maximum(v, Q_MIN), Q_MAX)
          outc_v[lax.div(i, 2), pl.ds(half + j * L, L)] = v * s
        return carry

      lax.fori_loop(0, _CH, row_body, 0, unroll=2)
      ostart = pl.multiple_of((base + cbase) // 2, 8)
      pltpu.sync_copy(outc_v, out_hbm.at[pl.ds(ostart, _CH // 2)])

  return dequant_gather


def kernel(x, weights, scales):
  V, D = weights.shape
  (B,) = x.shape
  table2 = weights.reshape(V // 2, 2 * D)
  out2 = _build(V, D, B)(table2, x.astype(jnp.int32), scales)
  return out2.reshape(B, D)
